# baseline (speedup 1.0000x reference)
import functools

import jax
import jax.numpy as jnp
from jax import lax
from jax.experimental import pallas as pl
from jax.experimental.pallas import tpu as pltpu
from jax.experimental.pallas import tpu_sc as plsc

_NC, _NS, _L = 2, 16, 16
_NW = _NC * _NS
_CROWS = 256
_NBUF = 3


def _make_sc_kernel(BH, KV, Q, D):
    rows_cur = BH * Q
    rpw = rows_cur // _NW
    bh_per_w = BH // _NW
    cache_rpw = BH * KV // _NW
    nchunks = cache_rpw // _CROWS
    mesh = plsc.VectorSubcoreMesh(core_axis_name="c", subcore_axis_name="s")

    @functools.partial(
        pl.kernel,
        mesh=mesh,
        out_type=jax.ShapeDtypeStruct((BH * KV, D), jnp.float32),
        scratch_types=[
            pltpu.VMEM((Q,), jnp.int32),
            pltpu.VMEM((rpw,), jnp.int32),
            pltpu.VMEM((rpw, D), jnp.float32),
            pltpu.VMEM((_NBUF, _CROWS, D), jnp.float32),
            pltpu.SemaphoreType.DMA((_NBUF,)),
            pltpu.SemaphoreType.DMA((_NBUF,)),
            pltpu.SemaphoreType.DMA,
            pltpu.SemaphoreType.DMA,
        ],
    )
    def sc_kernel(prev_hbm, cur_hbm, idx_hbm, out_hbm,
                  idx_v, dst_v, rows_v, bufs, insems, outsems, stage_sem, ssem):
        wid = lax.axis_index("s") * _NC + lax.axis_index("c")
        cbase = wid * cache_rpw

        base = wid * rpw
        stage = pltpu.make_async_copy(cur_hbm.at[pl.ds(base, rpw)], rows_v, stage_sem)
        stage.start()

        def in_cp(g, b):
            return pltpu.make_async_copy(
                prev_hbm.at[pl.ds(cbase + g * _CROWS, _CROWS)],
                bufs.at[b],
                insems.at[b],
            )

        def out_cp(g, b):
            return pltpu.make_async_copy(
                bufs.at[b],
                out_hbm.at[pl.ds(cbase + g * _CROWS, _CROWS)],
                outsems.at[b],
            )

        in_cp(0, 0).start()
        for g in range(nchunks):
            b = g % _NBUF
            nxt = g + 1
            if nxt < nchunks:
                bn = nxt % _NBUF
                if nxt >= _NBUF:
                    out_cp(nxt - _NBUF, bn).wait()
                in_cp(nxt, bn).start()
            in_cp(g, b).wait()
            out_cp(g, b).start()
        for g in range(max(0, nchunks - _NBUF), nchunks):
            out_cp(g, g % _NBUF).wait()

        pltpu.sync_copy(idx_hbm, idx_v)
        stage.wait()
        iv = idx_v[...]
        for j in range(bh_per_w):
            bh = wid * bh_per_w + j
            dst_v[pl.ds(j * _L, _L)] = iv + bh * KV
        sc = pltpu.make_async_copy(rows_v, out_hbm.at[dst_v], ssem)
        sc.start()
        sc.wait()

    return sc_kernel


def kernel(prev, cur, dim, idx, inp_seq_len):
    B, H, KV, D = prev.shape
    Q = cur.shape[2]
    idx = (idx + (jnp.asarray(dim, dtype=idx.dtype) - 2)).astype(jnp.int32)

    prev2 = prev.reshape(B * H * KV, D)
    cur2 = cur.reshape(B * H * Q, D)

    sc_kernel = _make_sc_kernel(B * H, KV, Q, D)
    out2 = sc_kernel(prev2, cur2, idx)
    return out2.reshape(B, H, KV, D)


# speedup vs baseline: 1.0049x; 1.0049x over previous
import functools

import jax
import jax.numpy as jnp
from jax import lax
from jax.experimental import pallas as pl
from jax.experimental.pallas import tpu as pltpu
from jax.experimental.pallas import tpu_sc as plsc

_NC, _NS, _L = 2, 16, 16
_NW = _NC * _NS
_CROWS = 128
_NBUF = 6
_LOOKAHEAD = 3


def _make_sc_kernel(BH, KV, Q, D):
    rows_cur = BH * Q
    rpw = rows_cur // _NW
    bh_per_w = BH // _NW
    cache_rpw = BH * KV // _NW
    nchunks = cache_rpw // _CROWS
    mesh = plsc.VectorSubcoreMesh(core_axis_name="c", subcore_axis_name="s")

    @functools.partial(
        pl.kernel,
        mesh=mesh,
        out_type=jax.ShapeDtypeStruct((BH * KV, D), jnp.float32),
        scratch_types=[
            pltpu.VMEM((Q,), jnp.int32),
            pltpu.VMEM((rpw,), jnp.int32),
            pltpu.VMEM((rpw, D), jnp.float32),
            pltpu.VMEM((_NBUF, _CROWS, D), jnp.float32),
            pltpu.SemaphoreType.DMA((_NBUF,)),
            pltpu.SemaphoreType.DMA((_NBUF,)),
            pltpu.SemaphoreType.DMA,
            pltpu.SemaphoreType.DMA,
        ],
    )
    def sc_kernel(prev_hbm, cur_hbm, idx_hbm, out_hbm,
                  idx_v, dst_v, rows_v, bufs, insems, outsems, stage_sem, ssem):
        wid = lax.axis_index("s") * _NC + lax.axis_index("c")
        cbase = wid * cache_rpw

        base = wid * rpw
        stage = pltpu.make_async_copy(cur_hbm.at[pl.ds(base, rpw)], rows_v, stage_sem)
        stage.start()

        def in_cp(g, b):
            return pltpu.make_async_copy(
                prev_hbm.at[pl.ds(cbase + g * _CROWS, _CROWS)],
                bufs.at[b],
                insems.at[b],
            )

        def out_cp(g, b):
            return pltpu.make_async_copy(
                bufs.at[b],
                out_hbm.at[pl.ds(cbase + g * _CROWS, _CROWS)],
                outsems.at[b],
            )

        for p in range(_LOOKAHEAD):
            in_cp(p, p % _NBUF).start()
        for g in range(nchunks):
            b = g % _NBUF
            nxt = g + _LOOKAHEAD
            if nxt < nchunks:
                bn = nxt % _NBUF
                if nxt >= _NBUF:
                    out_cp(nxt - _NBUF, bn).wait()
                in_cp(nxt, bn).start()
            in_cp(g, b).wait()
            out_cp(g, b).start()
        for g in range(max(0, nchunks - _NBUF), nchunks):
            out_cp(g, g % _NBUF).wait()

        pltpu.sync_copy(idx_hbm, idx_v)
        stage.wait()
        iv = idx_v[...]
        for j in range(bh_per_w):
            bh = wid * bh_per_w + j
            dst_v[pl.ds(j * _L, _L)] = iv + bh * KV
        sc = pltpu.make_async_copy(rows_v, out_hbm.at[dst_v], ssem)
        sc.start()
        sc.wait()

    return sc_kernel


def kernel(prev, cur, dim, idx, inp_seq_len):
    B, H, KV, D = prev.shape
    Q = cur.shape[2]
    idx = (idx + (jnp.asarray(dim, dtype=idx.dtype) - 2)).astype(jnp.int32)

    prev2 = prev.reshape(B * H * KV, D)
    cur2 = cur.reshape(B * H * Q, D)

    sc_kernel = _make_sc_kernel(B * H, KV, Q, D)
    out2 = sc_kernel(prev2, cur2, idx)
    return out2.reshape(B, H, KV, D)
